# baseline (speedup 1.0000x reference)
import functools

import jax
import jax.numpy as jnp
from jax import lax
from jax.experimental import pallas as pl
from jax.experimental.pallas import tpu as pltpu
from jax.experimental.pallas import tpu_sc as plsc

N = 10000
H = 128
E = 320000

_NS = 16
_CHUNK = 64
_CPT = 320
_EPT = _CPT * _CHUNK
_EPAD = _EPT * _NS
_RSLAB = 640
_GRP = 32
_NGRP = _CPT // _GRP
_NBUF = 4

_BN = 1000
_NB = N // _BN


def _mm(a, b):
    return lax.dot_general(a, b, (((1,), (0,)), ((), ())),
                           preferred_element_type=jnp.float32)


def _scale_chunk(rows, ev_v, c):
    def _scale(q2, c2):
        ev16 = ev_v[pl.ds(c * _CHUNK + q2 * 16, 16)]
        for t in range(16):
            evk = ev16[t]
            r = q2 * 16 + t
            for j2 in range(8):
                rows[r, pl.ds(j2 * 16, 16)] = (
                    rows[r, pl.ds(j2 * 16, 16)] * evk)
        return c2

    lax.fori_loop(0, _CHUNK // 16, _scale, 0)


def _segsum_body(s_hbm, src_hbm, dst_hbm, ev_hbm, m_hbm,
                 idx_v, ev_v, dst_v, rows0, rows1, rows2, rows3,
                 acc, gs0, gs1, gs2, gs3, ss0, ss1, ss2, ss3):
    cid = lax.axis_index("c")
    sid = lax.axis_index("s")
    rows = (rows0, rows1, rows2, rows3)
    gs = (gs0, gs1, gs2, gs3)
    ss = (ss0, ss1, ss2, ss3)

    zeros16 = jnp.zeros((16,), jnp.float32)

    def _zrow(r, carry):
        for j2 in range(8):
            rows0[r, pl.ds(j2 * 16, 16)] = zeros16
        return carry

    lax.fori_loop(0, _CHUNK, _zrow, 0)
    rbase = sid * _RSLAB

    @pl.when(sid < _NS - 1)
    def _():
        for q in range(10):
            pltpu.sync_copy(rows0, acc.at[pl.ds(rbase + q * 64, 64)])

    @pl.when(sid == _NS - 1)
    def _():
        for q in range(6):
            pltpu.sync_copy(rows0, acc.at[pl.ds(rbase + q * 64, 64)])
        pltpu.sync_copy(rows0.at[pl.ds(0, 16)],
                        acc.at[pl.ds(rbase + 384, 16)])

    plsc.subcore_barrier()

    def _group(g, carry):
        @pl.when(g > 0)
        def _():
            for x in range(_NBUF):
                pltpu.make_async_copy(rows[x], acc.at[dst_v.at[x]],
                                      ss[x]).wait()

        cb = sid * _CPT + g * _GRP
        pltpu.sync_copy(src_hbm.at[cid, pl.ds(cb, _GRP)], idx_v)
        pltpu.sync_copy(dst_hbm.at[cid, pl.ds(cb, _GRP)], dst_v)
        pltpu.sync_copy(ev_hbm.at[cid, pl.ds(cb * _CHUNK, _GRP * _CHUNK)],
                        ev_v)
        for x in range(_NBUF):
            pltpu.make_async_copy(s_hbm.at[idx_v.at[x]], rows[x],
                                  gs[x]).start()

        def _quad(q, c1):
            for x in range(_NBUF):
                c = _NBUF * q + x
                pltpu.make_async_copy(s_hbm.at[idx_v.at[c]], rows[x],
                                      gs[x]).wait()
                _scale_chunk(rows[x], ev_v, c)
                pltpu.make_async_copy(rows[x], acc.at[dst_v.at[c]],
                                      ss[x]).start(add=True)

            @pl.when(q < _GRP // _NBUF - 1)
            def _():
                for x in range(_NBUF):
                    c = _NBUF * q + x
                    pltpu.make_async_copy(rows[x], acc.at[dst_v.at[c]],
                                          ss[x]).wait()
                    pltpu.make_async_copy(s_hbm.at[idx_v.at[c + _NBUF]],
                                          rows[x], gs[x]).start()

            return c1

        lax.fori_loop(0, _GRP // _NBUF, _quad, 0)
        return carry

    lax.fori_loop(0, _NGRP, _group, 0)
    for x in range(_NBUF):
        pltpu.make_async_copy(rows[x], acc.at[dst_v.at[x]], ss[x]).wait()
    plsc.subcore_barrier()

    @pl.when(sid < _NS - 1)
    def _():
        for q in range(10):
            pltpu.sync_copy(acc.at[pl.ds(rbase + q * 64, 64)], rows0)
            pltpu.sync_copy(rows0, m_hbm.at[cid, pl.ds(rbase + q * 64, 64)])

    @pl.when(sid == _NS - 1)
    def _():
        for q in range(6):
            pltpu.sync_copy(acc.at[pl.ds(rbase + q * 64, 64)], rows0)
            pltpu.sync_copy(rows0, m_hbm.at[cid, pl.ds(rbase + q * 64, 64)])
        pltpu.sync_copy(acc.at[pl.ds(rbase + 384, 16)],
                        rows0.at[pl.ds(0, 16)])
        pltpu.sync_copy(rows0.at[pl.ds(0, 16)],
                        m_hbm.at[cid, pl.ds(rbase + 384, 16)])


def _segsum_sc(s2n, src, dst, ev):
    mesh = plsc.VectorSubcoreMesh(core_axis_name="c", subcore_axis_name="s",
                                  num_cores=2, num_subcores=_NS)
    return pl.kernel(
        _segsum_body,
        out_type=jax.ShapeDtypeStruct((2, N, H), jnp.float32),
        mesh=mesh,
        scratch_types=(
            [pltpu.VMEM((_GRP, _CHUNK), jnp.int32),
             pltpu.VMEM((_GRP * _CHUNK,), jnp.float32),
             pltpu.VMEM((_GRP, _CHUNK), jnp.int32)]
            + [pltpu.VMEM((_CHUNK, H), jnp.float32)] * _NBUF
            + [pltpu.VMEM_SHARED((N, H), jnp.float32)]
            + [pltpu.SemaphoreType.DMA] * (2 * _NBUF)
        ),
    )(s2n, src, dst, ev)


def _prep_body(x_ref, rw_ref, rb_ref, sU_ref, apw_ref, apb_ref, apu_ref,
               s_out, xU_out, pool_out):
    i = pl.program_id(0)
    xb = x_ref[...]
    s_out[0] = _mm(xb, rw_ref[0]) + rb_ref[0]
    s_out[1] = _mm(xb, rw_ref[1]) + rb_ref[1]
    xU_out[...] = _mm(xb, sU_ref[...])
    hh = jnp.tanh(_mm(xb, apw_ref[...]) + apb_ref[...])
    e = jnp.exp(jnp.sum(hh * apu_ref[...], axis=1, keepdims=True))

    @pl.when(i == 0)
    def _():
        pool_out[...] = jnp.zeros_like(pool_out)

    pool_out[0:1, :] += jnp.sum(e * xb, axis=0, keepdims=True)
    pool_out[1:2, :] += jnp.sum(e)


def _prep_call(x, rw, rb, sU, apw, apb, apu):
    return pl.pallas_call(
        _prep_body,
        grid=(_NB,),
        in_specs=[
            pl.BlockSpec((_BN, H), lambda i: (i, 0)),
            pl.BlockSpec((2, H, H), lambda i: (0, 0, 0)),
            pl.BlockSpec((2, 1, H), lambda i: (0, 0, 0)),
            pl.BlockSpec((H, 4 * H), lambda i: (0, 0)),
            pl.BlockSpec((H, H), lambda i: (0, 0)),
            pl.BlockSpec((1, H), lambda i: (0, 0)),
            pl.BlockSpec((1, H), lambda i: (0, 0)),
        ],
        out_specs=[
            pl.BlockSpec((2, _BN, H), lambda i: (0, i, 0)),
            pl.BlockSpec((_BN, 4 * H), lambda i: (i, 0)),
            pl.BlockSpec((8, H), lambda i: (0, 0)),
        ],
        out_shape=[
            jax.ShapeDtypeStruct((2, N, H), jnp.float32),
            jax.ShapeDtypeStruct((N, 4 * H), jnp.float32),
            jax.ShapeDtypeStruct((8, H), jnp.float32),
        ],
    )(x, rw, rb, sU, apw, apb, apu)


def _init_body(pool_ref, sV_ref, sVb_ref, gww_ref, gv_out, gV_out):
    g = pool_ref[0:1, :] / pool_ref[1:2, :]
    gv_out[...] = jnp.zeros_like(gv_out)
    gv_out[0:1, :] = g
    gv_out[1:2, :] = g
    gv_out[2:3, :] = _mm(g, gww_ref[...])
    gV_out[...] = jnp.zeros_like(gV_out)
    gV_out[0:1, :] = _mm(g, sV_ref[...]) + sVb_ref[...]


def _init_call(pool, sV, sVb, gww):
    return pl.pallas_call(
        _init_body,
        out_shape=[
            jax.ShapeDtypeStruct((8, H), jnp.float32),
            jax.ShapeDtypeStruct((8, 4 * H), jnp.float32),
        ],
    )(pool, sV, sVb, gww)


def _layer_body(m_ref, h_ref, c_ref, xU_ref, gv_ref, gV_ref,
                sWh_ref, sWn_ref, apw_ref, apb_ref, apu_ref,
                gu_ref, gub_ref, rw_ref, rb_ref,
                h_out, c_out, s_out, pool_out, fw_out, *, last):
    i = pl.program_id(0)
    hn = jnp.tanh(m_ref[0] + m_ref[1])
    hb = h_ref[...]
    gates = (_mm(hb, sWh_ref[...]) + _mm(hn, sWn_ref[...])
             + xU_ref[...] + gV_ref[0:1, :])
    ig = gates[:, 0:H]
    fg = gates[:, H:2 * H]
    og = gates[:, 2 * H:3 * H]
    ug = gates[:, 3 * H:4 * H]
    c2 = jax.nn.sigmoid(fg) * c_ref[...] + jax.nn.sigmoid(ig) * jnp.tanh(ug)
    h2 = jax.nn.sigmoid(og) * jnp.tanh(c2)
    h_out[...] = h2
    if last:
        return
    c_out[...] = c2
    s_out[0] = _mm(h2, rw_ref[0]) + rb_ref[0]
    s_out[1] = _mm(h2, rw_ref[1]) + rb_ref[1]
    hh = jnp.tanh(_mm(h2, apw_ref[...]) + apb_ref[...])
    e = jnp.exp(jnp.sum(hh * apu_ref[...], axis=1, keepdims=True))
    z = jax.nn.sigmoid(gv_ref[2:3, :] + _mm(h2, gu_ref[...]) + gub_ref[...])
    ez = jnp.exp(z)

    @pl.when(i == 0)
    def _():
        pool_out[...] = jnp.zeros_like(pool_out)
        fw_out[...] = jnp.zeros_like(fw_out)

    pool_out[0:1, :] += jnp.sum(e * h2, axis=0, keepdims=True)
    pool_out[1:2, :] += jnp.sum(e)
    fw_out[0:1, :] += jnp.sum(c2 * ez, axis=0, keepdims=True)
    fw_out[1:2, :] += jnp.sum(ez, axis=0, keepdims=True)


def _layer_call(m, h, c, xU, gv, gV, sWh, sWn, apw, apb, apu, gu, gub,
                rw, rb, last):
    small = lambda shape: pl.BlockSpec(shape, lambda i: tuple(0 for _ in shape))
    in_specs = [
        pl.BlockSpec((2, _BN, H), lambda i: (0, i, 0)),
        pl.BlockSpec((_BN, H), lambda i: (i, 0)),
        pl.BlockSpec((_BN, H), lambda i: (i, 0)),
        pl.BlockSpec((_BN, 4 * H), lambda i: (i, 0)),
        small((8, H)),
        small((8, 4 * H)),
        small((H, 4 * H)),
        small((H, 4 * H)),
        small((H, H)),
        small((1, H)),
        small((1, H)),
        small((H, H)),
        small((1, H)),
        small((2, H, H)),
        small((2, 1, H)),
    ]
    if last:
        out_specs = [pl.BlockSpec((_BN, H), lambda i: (i, 0))]
        out_shape = [jax.ShapeDtypeStruct((N, H), jnp.float32)]
        body = lambda *refs: _layer_body(
            *refs[:15], refs[15], None, None, None, None, last=True)
    else:
        out_specs = [
            pl.BlockSpec((_BN, H), lambda i: (i, 0)),
            pl.BlockSpec((_BN, H), lambda i: (i, 0)),
            pl.BlockSpec((2, _BN, H), lambda i: (0, i, 0)),
            pl.BlockSpec((8, H), lambda i: (0, 0)),
            pl.BlockSpec((8, H), lambda i: (0, 0)),
        ]
        out_shape = [
            jax.ShapeDtypeStruct((N, H), jnp.float32),
            jax.ShapeDtypeStruct((N, H), jnp.float32),
            jax.ShapeDtypeStruct((2, N, H), jnp.float32),
            jax.ShapeDtypeStruct((8, H), jnp.float32),
            jax.ShapeDtypeStruct((8, H), jnp.float32),
        ]
        body = functools.partial(_layer_body, last=False)
    return pl.pallas_call(
        body,
        grid=(_NB,),
        in_specs=in_specs,
        out_specs=out_specs,
        out_shape=out_shape,
    )(m, h, c, xU, gv, gV, sWh, sWn, apw, apb, apu, gu, gub, rw, rb)


def _update_body(pool_ref, fw_ref, gvp_ref, gW_ref, gU_ref, gUb_ref,
                 gww_ref, sV_ref, sVb_ref, gv_out, gV_out):
    h_avg = pool_ref[0:1, :] / pool_ref[1:2, :]
    g = gvp_ref[0:1, :]
    c_g = gvp_ref[1:2, :]
    fo = jax.nn.sigmoid(_mm(g, gW_ref[...]) + _mm(h_avg, gU_ref[...])
                        + gUb_ref[...])
    f2 = fo[:, 0:H]
    o2 = fo[:, H:2 * H]
    cg2 = f2 * c_g + fw_ref[0:1, :] / fw_ref[1:2, :]
    g2 = o2 * jnp.tanh(cg2)
    gv_out[...] = jnp.zeros_like(gv_out)
    gv_out[0:1, :] = g2
    gv_out[1:2, :] = cg2
    gv_out[2:3, :] = _mm(g2, gww_ref[...])
    gV_out[...] = jnp.zeros_like(gV_out)
    gV_out[0:1, :] = _mm(g2, sV_ref[...]) + sVb_ref[...]


def _update_call(pool, fw, gvp, gW, gU, gUb, gww, sV, sVb):
    return pl.pallas_call(
        _update_body,
        out_shape=[
            jax.ShapeDtypeStruct((8, H), jnp.float32),
            jax.ShapeDtypeStruct((8, 4 * H), jnp.float32),
        ],
    )(pool, fw, gvp, gW, gU, gUb, gww, sV, sVb)


def kernel(x, edge_index_r0, edge_val_r0, edge_index_r1, edge_val_r1,
           ap_w1, ap_b1, ap_u,
           s_Wh, s_Wn, s_U, s_V, s_V_b,
           rgcn_w0, rgcn_b0, rgcn_w1, rgcn_b1,
           g_W, g_w, g_U, g_U_b, g_u, g_u_b):
    pad = _EPAD - E
    src = jnp.stack([edge_index_r0[1], edge_index_r1[1] + N])
    src = jnp.pad(src, ((0, 0), (0, pad))).reshape(2, _EPAD // _CHUNK, _CHUNK)
    dst = jnp.stack([edge_index_r0[0], edge_index_r1[0]])
    dst = jnp.pad(dst, ((0, 0), (0, pad))).reshape(2, _EPAD // _CHUNK, _CHUNK)
    ev = jnp.stack([edge_val_r0, edge_val_r1])
    ev = jnp.pad(ev, ((0, 0), (0, pad)))

    rw = jnp.stack([rgcn_w0, rgcn_w1])
    rb = jnp.stack([rgcn_b0, rgcn_b1]).reshape(2, 1, H)
    apb = ap_b1.reshape(1, H)
    apu = ap_u.reshape(1, H)
    gub = g_u_b.reshape(1, H)
    gUb = g_U_b.reshape(1, 2 * H)
    sVb = s_V_b.reshape(1, 4 * H)

    s_stack, xU, pool0 = _prep_call(x, rw, rb, s_U, ap_w1, apb, apu)
    gv, gV = _init_call(pool0, s_V, sVb, g_w)

    h, c = x, x
    for layer in range(2):
        m = _segsum_sc(s_stack.reshape(2 * N, H), src, dst, ev)
        last = layer == 1
        if last:
            (h,) = _layer_call(m, h, c, xU, gv, gV, s_Wh, s_Wn, ap_w1, apb,
                               apu, g_u, gub, rw, rb, last=True)
        else:
            h, c, s_stack, pool, fw = _layer_call(
                m, h, c, xU, gv, gV, s_Wh, s_Wn, ap_w1, apb, apu, g_u, gub,
                rw, rb, last=False)
            gv, gV = _update_call(pool, fw, gv, g_W, g_U, gUb, g_w, s_V, sVb)
    return h


# speedup vs baseline: 1.0195x; 1.0195x over previous
import functools

import jax
import jax.numpy as jnp
from jax import lax
from jax.experimental import pallas as pl
from jax.experimental.pallas import tpu as pltpu
from jax.experimental.pallas import tpu_sc as plsc

N = 10000
H = 128
E = 320000

_NS = 16
_CHUNK = 128
_CPT = 160
_EPT = _CPT * _CHUNK
_EPAD = _EPT * _NS
_RSLAB = 640
_GRP = 32
_NGRP = _CPT // _GRP

_BN = 1000
_NB = N // _BN


def _mm(a, b):
    return lax.dot_general(a, b, (((1,), (0,)), ((), ())),
                           preferred_element_type=jnp.float32)


def _scale_chunk(rows, ev_v, c):
    def _scale(q2, c2):
        ev16 = ev_v[pl.ds(c * _CHUNK + q2 * 16, 16)]
        for t in range(16):
            evk = ev16[t]
            r = q2 * 16 + t
            for j2 in range(8):
                rows[r, pl.ds(j2 * 16, 16)] = (
                    rows[r, pl.ds(j2 * 16, 16)] * evk)
        return c2

    lax.fori_loop(0, _CHUNK // 16, _scale, 0)


def _segsum_body(s_hbm, src_hbm, dst_hbm, ev_hbm, m_hbm,
                 idx_v, ev_v, dst_v, rows_a, rows_b, acc,
                 gsa, gsb, ssa, ssb):
    cid = lax.axis_index("c")
    sid = lax.axis_index("s")

    zeros16 = jnp.zeros((16,), jnp.float32)

    def _zrow(r, carry):
        for j2 in range(8):
            rows_a[r, pl.ds(j2 * 16, 16)] = zeros16
        return carry

    lax.fori_loop(0, _CHUNK, _zrow, 0)
    rbase = sid * _RSLAB

    @pl.when(sid < _NS - 1)
    def _():
        for q in range(5):
            pltpu.sync_copy(rows_a, acc.at[pl.ds(rbase + q * 128, 128)])

    @pl.when(sid == _NS - 1)
    def _():
        for q in range(3):
            pltpu.sync_copy(rows_a, acc.at[pl.ds(rbase + q * 128, 128)])
        pltpu.sync_copy(rows_a.at[pl.ds(0, 16)],
                        acc.at[pl.ds(rbase + 384, 16)])

    plsc.subcore_barrier()

    def _group(g, carry):
        cb = sid * _CPT + g * _GRP
        pltpu.sync_copy(src_hbm.at[cid, pl.ds(cb, _GRP)], idx_v)

        @pl.when(g > 0)
        def _():
            pltpu.make_async_copy(rows_a, acc.at[dst_v.at[0]], ssa).wait()
            pltpu.make_async_copy(rows_b, acc.at[dst_v.at[1]], ssb).wait()

        pltpu.make_async_copy(s_hbm.at[idx_v.at[0]], rows_a, gsa).start()
        pltpu.make_async_copy(s_hbm.at[idx_v.at[1]], rows_b, gsb).start()
        pltpu.sync_copy(dst_hbm.at[cid, pl.ds(cb, _GRP)], dst_v)
        pltpu.sync_copy(ev_hbm.at[cid, pl.ds(cb * _CHUNK, _GRP * _CHUNK)],
                        ev_v)

        def _pair(q, c1):
            ca = 2 * q
            cb2 = 2 * q + 1
            pltpu.make_async_copy(s_hbm.at[idx_v.at[ca]], rows_a, gsa).wait()
            _scale_chunk(rows_a, ev_v, ca)
            pltpu.make_async_copy(rows_a, acc.at[dst_v.at[ca]],
                                  ssa).start(add=True)
            pltpu.make_async_copy(s_hbm.at[idx_v.at[cb2]], rows_b, gsb).wait()
            _scale_chunk(rows_b, ev_v, cb2)
            pltpu.make_async_copy(rows_b, acc.at[dst_v.at[cb2]],
                                  ssb).start(add=True)

            @pl.when(q < _GRP // 2 - 1)
            def _():
                pltpu.make_async_copy(rows_a, acc.at[dst_v.at[ca]],
                                      ssa).wait()
                pltpu.make_async_copy(s_hbm.at[idx_v.at[ca + 2]], rows_a,
                                      gsa).start()
                pltpu.make_async_copy(rows_b, acc.at[dst_v.at[cb2]],
                                      ssb).wait()
                pltpu.make_async_copy(s_hbm.at[idx_v.at[cb2 + 2]], rows_b,
                                      gsb).start()

            return c1

        lax.fori_loop(0, _GRP // 2, _pair, 0)
        return carry

    lax.fori_loop(0, _NGRP, _group, 0)
    pltpu.make_async_copy(rows_a, acc.at[dst_v.at[0]], ssa).wait()
    pltpu.make_async_copy(rows_b, acc.at[dst_v.at[1]], ssb).wait()
    plsc.subcore_barrier()

    @pl.when(sid < _NS - 1)
    def _():
        for q in range(5):
            pltpu.sync_copy(acc.at[pl.ds(rbase + q * 128, 128)], rows_a)
            pltpu.sync_copy(rows_a, m_hbm.at[cid, pl.ds(rbase + q * 128, 128)])

    @pl.when(sid == _NS - 1)
    def _():
        for q in range(3):
            pltpu.sync_copy(acc.at[pl.ds(rbase + q * 128, 128)], rows_a)
            pltpu.sync_copy(rows_a, m_hbm.at[cid, pl.ds(rbase + q * 128, 128)])
        pltpu.sync_copy(acc.at[pl.ds(rbase + 384, 16)],
                        rows_a.at[pl.ds(0, 16)])
        pltpu.sync_copy(rows_a.at[pl.ds(0, 16)],
                        m_hbm.at[cid, pl.ds(rbase + 384, 16)])


def _segsum_sc(s2n, src, dst, ev):
    mesh = plsc.VectorSubcoreMesh(core_axis_name="c", subcore_axis_name="s",
                                  num_cores=2, num_subcores=_NS)
    return pl.kernel(
        _segsum_body,
        out_type=jax.ShapeDtypeStruct((2, N, H), jnp.float32),
        mesh=mesh,
        scratch_types=[
            pltpu.VMEM((_GRP, _CHUNK), jnp.int32),
            pltpu.VMEM((_GRP * _CHUNK,), jnp.float32),
            pltpu.VMEM((_GRP, _CHUNK), jnp.int32),
            pltpu.VMEM((_CHUNK, H), jnp.float32),
            pltpu.VMEM((_CHUNK, H), jnp.float32),
            pltpu.VMEM_SHARED((N, H), jnp.float32),
            pltpu.SemaphoreType.DMA,
            pltpu.SemaphoreType.DMA,
            pltpu.SemaphoreType.DMA,
            pltpu.SemaphoreType.DMA,
        ],
    )(s2n, src, dst, ev)


def _prep_body(x_ref, rw_ref, rb_ref, sU_ref, apw_ref, apb_ref, apu_ref,
               s_out, xU_out, pool_out):
    i = pl.program_id(0)
    xb = x_ref[...]
    s_out[0] = _mm(xb, rw_ref[0]) + rb_ref[0]
    s_out[1] = _mm(xb, rw_ref[1]) + rb_ref[1]
    xU_out[...] = _mm(xb, sU_ref[...])
    hh = jnp.tanh(_mm(xb, apw_ref[...]) + apb_ref[...])
    e = jnp.exp(jnp.sum(hh * apu_ref[...], axis=1, keepdims=True))

    @pl.when(i == 0)
    def _():
        pool_out[...] = jnp.zeros_like(pool_out)

    pool_out[0:1, :] += jnp.sum(e * xb, axis=0, keepdims=True)
    pool_out[1:2, :] += jnp.sum(e)


def _prep_call(x, rw, rb, sU, apw, apb, apu):
    return pl.pallas_call(
        _prep_body,
        grid=(_NB,),
        in_specs=[
            pl.BlockSpec((_BN, H), lambda i: (i, 0)),
            pl.BlockSpec((2, H, H), lambda i: (0, 0, 0)),
            pl.BlockSpec((2, 1, H), lambda i: (0, 0, 0)),
            pl.BlockSpec((H, 4 * H), lambda i: (0, 0)),
            pl.BlockSpec((H, H), lambda i: (0, 0)),
            pl.BlockSpec((1, H), lambda i: (0, 0)),
            pl.BlockSpec((1, H), lambda i: (0, 0)),
        ],
        out_specs=[
            pl.BlockSpec((2, _BN, H), lambda i: (0, i, 0)),
            pl.BlockSpec((_BN, 4 * H), lambda i: (i, 0)),
            pl.BlockSpec((8, H), lambda i: (0, 0)),
        ],
        out_shape=[
            jax.ShapeDtypeStruct((2, N, H), jnp.float32),
            jax.ShapeDtypeStruct((N, 4 * H), jnp.float32),
            jax.ShapeDtypeStruct((8, H), jnp.float32),
        ],
    )(x, rw, rb, sU, apw, apb, apu)


def _init_body(pool_ref, sV_ref, sVb_ref, gww_ref, gv_out, gV_out):
    g = pool_ref[0:1, :] / pool_ref[1:2, :]
    gv_out[...] = jnp.zeros_like(gv_out)
    gv_out[0:1, :] = g
    gv_out[1:2, :] = g
    gv_out[2:3, :] = _mm(g, gww_ref[...])
    gV_out[...] = jnp.zeros_like(gV_out)
    gV_out[0:1, :] = _mm(g, sV_ref[...]) + sVb_ref[...]


def _init_call(pool, sV, sVb, gww):
    return pl.pallas_call(
        _init_body,
        out_shape=[
            jax.ShapeDtypeStruct((8, H), jnp.float32),
            jax.ShapeDtypeStruct((8, 4 * H), jnp.float32),
        ],
    )(pool, sV, sVb, gww)


def _layer_body(m_ref, h_ref, c_ref, xU_ref, gv_ref, gV_ref,
                sWh_ref, sWn_ref, apw_ref, apb_ref, apu_ref,
                gu_ref, gub_ref, rw_ref, rb_ref,
                h_out, c_out, s_out, pool_out, fw_out, *, last):
    i = pl.program_id(0)
    hn = jnp.tanh(m_ref[0] + m_ref[1])
    hb = h_ref[...]
    gates = (_mm(hb, sWh_ref[...]) + _mm(hn, sWn_ref[...])
             + xU_ref[...] + gV_ref[0:1, :])
    ig = gates[:, 0:H]
    fg = gates[:, H:2 * H]
    og = gates[:, 2 * H:3 * H]
    ug = gates[:, 3 * H:4 * H]
    c2 = jax.nn.sigmoid(fg) * c_ref[...] + jax.nn.sigmoid(ig) * jnp.tanh(ug)
    h2 = jax.nn.sigmoid(og) * jnp.tanh(c2)
    h_out[...] = h2
    if last:
        return
    c_out[...] = c2
    s_out[0] = _mm(h2, rw_ref[0]) + rb_ref[0]
    s_out[1] = _mm(h2, rw_ref[1]) + rb_ref[1]
    hh = jnp.tanh(_mm(h2, apw_ref[...]) + apb_ref[...])
    e = jnp.exp(jnp.sum(hh * apu_ref[...], axis=1, keepdims=True))
    z = jax.nn.sigmoid(gv_ref[2:3, :] + _mm(h2, gu_ref[...]) + gub_ref[...])
    ez = jnp.exp(z)

    @pl.when(i == 0)
    def _():
        pool_out[...] = jnp.zeros_like(pool_out)
        fw_out[...] = jnp.zeros_like(fw_out)

    pool_out[0:1, :] += jnp.sum(e * h2, axis=0, keepdims=True)
    pool_out[1:2, :] += jnp.sum(e)
    fw_out[0:1, :] += jnp.sum(c2 * ez, axis=0, keepdims=True)
    fw_out[1:2, :] += jnp.sum(ez, axis=0, keepdims=True)


def _layer_call(m, h, c, xU, gv, gV, sWh, sWn, apw, apb, apu, gu, gub,
                rw, rb, last):
    small = lambda shape: pl.BlockSpec(shape, lambda i: tuple(0 for _ in shape))
    in_specs = [
        pl.BlockSpec((2, _BN, H), lambda i: (0, i, 0)),
        pl.BlockSpec((_BN, H), lambda i: (i, 0)),
        pl.BlockSpec((_BN, H), lambda i: (i, 0)),
        pl.BlockSpec((_BN, 4 * H), lambda i: (i, 0)),
        small((8, H)),
        small((8, 4 * H)),
        small((H, 4 * H)),
        small((H, 4 * H)),
        small((H, H)),
        small((1, H)),
        small((1, H)),
        small((H, H)),
        small((1, H)),
        small((2, H, H)),
        small((2, 1, H)),
    ]
    if last:
        out_specs = [pl.BlockSpec((_BN, H), lambda i: (i, 0))]
        out_shape = [jax.ShapeDtypeStruct((N, H), jnp.float32)]
        body = lambda *refs: _layer_body(
            *refs[:15], refs[15], None, None, None, None, last=True)
    else:
        out_specs = [
            pl.BlockSpec((_BN, H), lambda i: (i, 0)),
            pl.BlockSpec((_BN, H), lambda i: (i, 0)),
            pl.BlockSpec((2, _BN, H), lambda i: (0, i, 0)),
            pl.BlockSpec((8, H), lambda i: (0, 0)),
            pl.BlockSpec((8, H), lambda i: (0, 0)),
        ]
        out_shape = [
            jax.ShapeDtypeStruct((N, H), jnp.float32),
            jax.ShapeDtypeStruct((N, H), jnp.float32),
            jax.ShapeDtypeStruct((2, N, H), jnp.float32),
            jax.ShapeDtypeStruct((8, H), jnp.float32),
            jax.ShapeDtypeStruct((8, H), jnp.float32),
        ]
        body = functools.partial(_layer_body, last=False)
    return pl.pallas_call(
        body,
        grid=(_NB,),
        in_specs=in_specs,
        out_specs=out_specs,
        out_shape=out_shape,
    )(m, h, c, xU, gv, gV, sWh, sWn, apw, apb, apu, gu, gub, rw, rb)


def _update_body(pool_ref, fw_ref, gvp_ref, gW_ref, gU_ref, gUb_ref,
                 gww_ref, sV_ref, sVb_ref, gv_out, gV_out):
    h_avg = pool_ref[0:1, :] / pool_ref[1:2, :]
    g = gvp_ref[0:1, :]
    c_g = gvp_ref[1:2, :]
    fo = jax.nn.sigmoid(_mm(g, gW_ref[...]) + _mm(h_avg, gU_ref[...])
                        + gUb_ref[...])
    f2 = fo[:, 0:H]
    o2 = fo[:, H:2 * H]
    cg2 = f2 * c_g + fw_ref[0:1, :] / fw_ref[1:2, :]
    g2 = o2 * jnp.tanh(cg2)
    gv_out[...] = jnp.zeros_like(gv_out)
    gv_out[0:1, :] = g2
    gv_out[1:2, :] = cg2
    gv_out[2:3, :] = _mm(g2, gww_ref[...])
    gV_out[...] = jnp.zeros_like(gV_out)
    gV_out[0:1, :] = _mm(g2, sV_ref[...]) + sVb_ref[...]


def _update_call(pool, fw, gvp, gW, gU, gUb, gww, sV, sVb):
    return pl.pallas_call(
        _update_body,
        out_shape=[
            jax.ShapeDtypeStruct((8, H), jnp.float32),
            jax.ShapeDtypeStruct((8, 4 * H), jnp.float32),
        ],
    )(pool, fw, gvp, gW, gU, gUb, gww, sV, sVb)


def kernel(x, edge_index_r0, edge_val_r0, edge_index_r1, edge_val_r1,
           ap_w1, ap_b1, ap_u,
           s_Wh, s_Wn, s_U, s_V, s_V_b,
           rgcn_w0, rgcn_b0, rgcn_w1, rgcn_b1,
           g_W, g_w, g_U, g_U_b, g_u, g_u_b):
    pad = _EPAD - E
    src = jnp.stack([edge_index_r0[1], edge_index_r1[1] + N])
    src = jnp.pad(src, ((0, 0), (0, pad))).reshape(2, _EPAD // _CHUNK, _CHUNK)
    dst = jnp.stack([edge_index_r0[0], edge_index_r1[0]])
    dst = jnp.pad(dst, ((0, 0), (0, pad))).reshape(2, _EPAD // _CHUNK, _CHUNK)
    ev = jnp.stack([edge_val_r0, edge_val_r1])
    ev = jnp.pad(ev, ((0, 0), (0, pad)))

    rw = jnp.stack([rgcn_w0, rgcn_w1])
    rb = jnp.stack([rgcn_b0, rgcn_b1]).reshape(2, 1, H)
    apb = ap_b1.reshape(1, H)
    apu = ap_u.reshape(1, H)
    gub = g_u_b.reshape(1, H)
    gUb = g_U_b.reshape(1, 2 * H)
    sVb = s_V_b.reshape(1, 4 * H)

    s_stack, xU, pool0 = _prep_call(x, rw, rb, s_U, ap_w1, apb, apu)
    gv, gV = _init_call(pool0, s_V, sVb, g_w)

    h, c = x, x
    for layer in range(2):
        m = _segsum_sc(s_stack.reshape(2 * N, H), src, dst, ev)
        last = layer == 1
        if last:
            (h,) = _layer_call(m, h, c, xU, gv, gV, s_Wh, s_Wn, ap_w1, apb,
                               apu, g_u, gub, rw, rb, last=True)
        else:
            h, c, s_stack, pool, fw = _layer_call(
                m, h, c, xU, gv, gV, s_Wh, s_Wn, ap_w1, apb, apu, g_u, gub,
                rw, rb, last=False)
            gv, gV = _update_call(pool, fw, gv, g_W, g_U, gUb, g_w, s_V, sVb)
    return h
